# baseline (speedup 1.0000x reference)
import functools

import jax
import jax.numpy as jnp
from jax.experimental import pallas as pl
from jax.experimental.pallas import tpu as pltpu


def _sage_kernel(x_ref, adj_ref, w_ref, b_ref, z_ref, s_ref, xw_ref, *, H):
    @pl.when(pl.program_id(1) == 0)
    def _():
        xw_ref[...] = jnp.einsum('bnf,fc->bnc', x_ref[...], w_ref[...],
                                 preferred_element_type=jnp.float32)

    h = jnp.einsum('bmn,bnc->bmc', adj_ref[...], xw_ref[...],
                   preferred_element_type=jnp.float32) + b_ref[...]
    he = h[:, :, :H]
    ha = h[:, :, H:]

    def norm_relu(t):
        ssq = jnp.sum(t * t, axis=-1, keepdims=True)
        t = t * jax.lax.rsqrt(jnp.maximum(ssq, 1e-24))
        return jnp.maximum(t, 0.0)

    he = norm_relu(he)
    ha = norm_relu(ha)

    def bn_stats(t):
        cnt = jnp.float32(t.shape[0] * t.shape[2])
        s1 = jnp.sum(jnp.sum(t, axis=2, keepdims=True), axis=0, keepdims=True)
        mean = s1 / cnt
        s2 = jnp.sum(jnp.sum(t * t, axis=2, keepdims=True), axis=0, keepdims=True)
        var = s2 / cnt - mean * mean
        inv = jax.lax.rsqrt(var + 1e-5)
        return mean, inv

    mean_e, inv_e = bn_stats(he)
    z_ref[...] = (he - mean_e) * inv_e

    _, inv_a = bn_stats(ha)
    t = ha * inv_a
    mx = jnp.max(t, axis=-1, keepdims=True)
    e = jnp.exp(t - mx)
    denom = jnp.sum(e, axis=-1, keepdims=True)
    s_ref[...] = e * pl.reciprocal(denom, approx=True)


def _pool_kernel(s_ref, z_ref, adj_ref, xnext_ref, anext_ref, *, K):
    sb = s_ref[...].astype(jnp.bfloat16)
    zb = z_ref[...].astype(jnp.bfloat16)
    a = adj_ref[...].astype(jnp.bfloat16)
    q = jnp.dot(a, sb, preferred_element_type=jnp.float32)
    qz = jnp.concatenate([q.astype(jnp.bfloat16), zb], axis=-1)
    out = jax.lax.dot_general(sb, qz, (((0,), (0,)), ((), ())),
                              preferred_element_type=jnp.float32)
    anext_ref[...] = out[:, :K]
    xnext_ref[...] = out[:, K:]


def kernel(x, adj, w_embed, b_embed, w_assign, b_assign):
    B, N, F = x.shape
    H = w_embed.shape[1]
    K = w_assign.shape[1]
    C = H + K

    wcat = jnp.concatenate([w_embed, w_assign], axis=1).astype(jnp.float32)
    bcat = jnp.concatenate([b_embed, b_assign]).reshape(1, 1, C).astype(jnp.float32)

    tn = 64 if N % 256 == 0 else N
    inner = max(N // (2 * tn), 1)
    tile = lambda i, j, _in=inner: (0, i * _in + j, 0)
    z_l, s_l = pl.pallas_call(
        functools.partial(_sage_kernel, H=H),
        out_shape=(jax.ShapeDtypeStruct((B, N, H), jnp.float32),
                   jax.ShapeDtypeStruct((B, N, K), jnp.float32)),
        grid=(N // (tn * inner), inner),
        in_specs=[pl.BlockSpec((B, N, F), lambda i, j: (0, 0, 0)),
                  pl.BlockSpec((B, tn, N), tile),
                  pl.BlockSpec((F, C), lambda i, j: (0, 0)),
                  pl.BlockSpec((1, 1, C), lambda i, j: (0, 0, 0))],
        out_specs=(pl.BlockSpec((B, tn, H), tile),
                   pl.BlockSpec((B, tn, K), tile)),
        scratch_shapes=[pltpu.VMEM((B, N, C), jnp.float32)],
        compiler_params=pltpu.CompilerParams(
            dimension_semantics=("parallel", "arbitrary"),
            vmem_limit_bytes=60 * 1024 * 1024),
        cost_estimate=pl.CostEstimate(
            flops=2 * B * N * N * F + 2 * B * N * F * C + 12 * B * N * C,
            transcendentals=B * N * (K + 4),
            bytes_accessed=4 * (B * N * N + B * N * F + 2 * B * N * C)),
    )(x, adj, wcat, bcat)

    xnext, anext = pl.pallas_call(
        functools.partial(_pool_kernel, K=K),
        out_shape=(jax.ShapeDtypeStruct((B, K, H), jnp.float32),
                   jax.ShapeDtypeStruct((B, K, K), jnp.float32)),
        grid=(B,),
        in_specs=[pl.BlockSpec((None, N, K), lambda b: (b, 0, 0)),
                  pl.BlockSpec((None, N, H), lambda b: (b, 0, 0)),
                  pl.BlockSpec((None, N, N), lambda b: (b, 0, 0))],
        out_specs=(pl.BlockSpec((None, K, H), lambda b: (b, 0, 0)),
                   pl.BlockSpec((None, K, K), lambda b: (b, 0, 0))),
        compiler_params=pltpu.CompilerParams(
            dimension_semantics=("parallel",),
            vmem_limit_bytes=60 * 1024 * 1024),
        cost_estimate=pl.CostEstimate(
            flops=2 * B * N * K * (N + K + H),
            transcendentals=0,
            bytes_accessed=4 * (B * N * (K + H + N) + B * K * (H + K))),
    )(s_l, z_l, adj)

    return xnext, anext, s_l, z_l


# speedup vs baseline: 1.1583x; 1.1583x over previous
import functools

import jax
import jax.numpy as jnp
from jax.experimental import pallas as pl
from jax.experimental.pallas import tpu as pltpu


def _sage_kernel(x_ref, adj_ref, w_ref, b_ref, z_ref, s_ref, xw_ref, *, H):
    @pl.when(pl.program_id(1) == 0)
    def _():
        xw_ref[...] = jnp.einsum('bnf,fc->bnc', x_ref[...], w_ref[...],
                                 preferred_element_type=jnp.float32)

    h = jnp.einsum('bmn,bnc->bmc', adj_ref[...], xw_ref[...],
                   preferred_element_type=jnp.float32) + b_ref[...]
    he = h[:, :, :H]
    ha = h[:, :, H:]

    def norm_relu(t):
        ssq = jnp.sum(t * t, axis=-1, keepdims=True)
        t = t * jax.lax.rsqrt(jnp.maximum(ssq, 1e-24))
        return jnp.maximum(t, 0.0)

    he = norm_relu(he)
    ha = norm_relu(ha)

    def bn_stats(t):
        cnt = jnp.float32(t.shape[0] * t.shape[2])
        s1 = jnp.sum(jnp.sum(t, axis=2, keepdims=True), axis=0, keepdims=True)
        mean = s1 / cnt
        s2 = jnp.sum(jnp.sum(t * t, axis=2, keepdims=True), axis=0, keepdims=True)
        var = s2 / cnt - mean * mean
        inv = jax.lax.rsqrt(var + 1e-5)
        return mean, inv

    mean_e, inv_e = bn_stats(he)
    z_ref[...] = (he - mean_e) * inv_e

    _, inv_a = bn_stats(ha)
    t = ha * inv_a
    mx = jnp.max(t, axis=-1, keepdims=True)
    e = jnp.exp(t - mx)
    denom = jnp.sum(e, axis=-1, keepdims=True)
    s_ref[...] = e * pl.reciprocal(denom, approx=True)


def _pool_kernel(s_ref, z_ref, adj_ref, xnext_ref, anext_ref, *, K):
    sb = s_ref[...].astype(jnp.bfloat16)
    zb = z_ref[...].astype(jnp.bfloat16)
    a = adj_ref[...].astype(jnp.bfloat16)
    q = jnp.einsum('gnm,gmk->gnk', a, sb,
                   preferred_element_type=jnp.float32)
    qz = jnp.concatenate([q.astype(jnp.bfloat16), zb], axis=-1)
    out = jnp.einsum('gnk,gnc->gkc', sb, qz,
                     preferred_element_type=jnp.float32)
    anext_ref[...] = out[:, :, :K]
    xnext_ref[...] = out[:, :, K:]


def kernel(x, adj, w_embed, b_embed, w_assign, b_assign):
    B, N, F = x.shape
    H = w_embed.shape[1]
    K = w_assign.shape[1]
    C = H + K

    wcat = jnp.concatenate([w_embed, w_assign], axis=1).astype(jnp.float32)
    bcat = jnp.concatenate([b_embed, b_assign]).reshape(1, 1, C).astype(jnp.float32)

    tn = 128 if N % 256 == 0 else N
    inner = max(N // (2 * tn), 1)
    tile = lambda i, j, _in=inner: (0, i * _in + j, 0)
    z_l, s_l = pl.pallas_call(
        functools.partial(_sage_kernel, H=H),
        out_shape=(jax.ShapeDtypeStruct((B, N, H), jnp.float32),
                   jax.ShapeDtypeStruct((B, N, K), jnp.float32)),
        grid=(N // (tn * inner), inner),
        in_specs=[pl.BlockSpec((B, N, F), lambda i, j: (0, 0, 0)),
                  pl.BlockSpec((B, tn, N), tile),
                  pl.BlockSpec((F, C), lambda i, j: (0, 0)),
                  pl.BlockSpec((1, 1, C), lambda i, j: (0, 0, 0))],
        out_specs=(pl.BlockSpec((B, tn, H), tile),
                   pl.BlockSpec((B, tn, K), tile)),
        scratch_shapes=[pltpu.VMEM((B, N, C), jnp.float32)],
        compiler_params=pltpu.CompilerParams(
            dimension_semantics=("parallel", "arbitrary"),
            vmem_limit_bytes=60 * 1024 * 1024),
        cost_estimate=pl.CostEstimate(
            flops=2 * B * N * N * F + 2 * B * N * F * C + 12 * B * N * C,
            transcendentals=B * N * (K + 4),
            bytes_accessed=4 * (B * N * N + B * N * F + 2 * B * N * C)),
    )(x, adj, wcat, bcat)

    G = 2 if B % 4 == 0 else 1
    xnext, anext = pl.pallas_call(
        functools.partial(_pool_kernel, K=K),
        out_shape=(jax.ShapeDtypeStruct((B, K, H), jnp.float32),
                   jax.ShapeDtypeStruct((B, K, K), jnp.float32)),
        grid=(B // G,),
        in_specs=[pl.BlockSpec((G, N, K), lambda b: (b, 0, 0)),
                  pl.BlockSpec((G, N, H), lambda b: (b, 0, 0)),
                  pl.BlockSpec((G, N, N), lambda b: (b, 0, 0))],
        out_specs=(pl.BlockSpec((G, K, H), lambda b: (b, 0, 0)),
                   pl.BlockSpec((G, K, K), lambda b: (b, 0, 0))),
        compiler_params=pltpu.CompilerParams(
            dimension_semantics=("parallel",),
            vmem_limit_bytes=60 * 1024 * 1024),
        cost_estimate=pl.CostEstimate(
            flops=2 * B * N * K * (N + K + H),
            transcendentals=0,
            bytes_accessed=4 * (B * N * (K + H + N) + B * K * (H + K))),
    )(s_l, z_l, adj)

    return xnext, anext, s_l, z_l
